# Initial kernel scaffold; baseline (speedup 1.0000x reference)
import jax
import jax.numpy as jnp
from jax import lax
from jax.experimental import pallas as pl
from jax.experimental.pallas import tpu as pltpu
from jax.experimental.pallas import tpu_sc as plsc

N = 10000
D = 128
E = 320000
B = 10000

NC, NS, L = 2, 16, 16
NW = NC * NS
N_PAD = 10240
ROWS_PER_TILE = N_PAD // NS
E_PAD = NW * 80 * 128
EG = 80
B_PAD = NW * 3 * 128
BG = 3

_MESH = plsc.VectorSubcoreMesh(
    core_axis_name="c", subcore_axis_name="s", num_cores=NC, num_subcores=NS
)


def _edge_body(src_hbm, dst_hbm, feat_hbm, agg_hbm, deg_hbm,
               srcb, dstb, rows, ones, zdeg, sem, agg_sh, deg_sh):
    cid = lax.axis_index("c")
    sid = lax.axis_index("s")
    wid = sid * NC + cid
    row0 = sid * ROWS_PER_TILE

    zf = jnp.zeros((L,), jnp.float32)
    of = jnp.ones((L,), jnp.float32)

    def _zrows(i, _):
        r = i // 8
        c = (i % 8) * L
        rows[r, pl.ds(c, L)] = zf
        return 0
    lax.fori_loop(0, 128 * 8, _zrows, 0)

    def _zdeg(i, _):
        zdeg[i, :] = zf
        return 0
    lax.fori_loop(0, ROWS_PER_TILE, _zdeg, 0)

    def _ones(i, _):
        ones[i, :] = of
        return 0
    lax.fori_loop(0, 128, _ones, 0)

    for k in range(ROWS_PER_TILE // 128):
        pltpu.sync_copy(rows, agg_sh.at[pl.ds(row0 + k * 128, 128)])
    pltpu.sync_copy(zdeg, deg_sh.at[pl.ds(row0, ROWS_PER_TILE)])
    plsc.subcore_barrier()

    pltpu.sync_copy(src_hbm.at[wid], srcb)
    pltpu.sync_copy(dst_hbm.at[wid], dstb)

    def _edges(j, _):
        pltpu.async_copy(feat_hbm.at[srcb.at[j]], rows, sem).wait()
        pltpu.sync_copy(rows, agg_sh.at[dstb.at[j]], add=True)
        pltpu.sync_copy(ones, deg_sh.at[dstb.at[j]], add=True)
        return 0
    lax.fori_loop(0, EG, _edges, 0)

    plsc.subcore_barrier()

    out0 = cid * N_PAD + row0
    pltpu.sync_copy(agg_sh.at[pl.ds(row0, ROWS_PER_TILE)],
                    agg_hbm.at[pl.ds(out0, ROWS_PER_TILE)])
    pltpu.sync_copy(deg_sh.at[pl.ds(row0, ROWS_PER_TILE)],
                    deg_hbm.at[pl.ds(out0, ROWS_PER_TILE)])


_edge_kernel = pl.kernel(
    _edge_body,
    out_type=(
        jax.ShapeDtypeStruct((NC * N_PAD, D), jnp.float32),
        jax.ShapeDtypeStruct((NC * N_PAD, L), jnp.float32),
    ),
    mesh=_MESH,
    scratch_types=[
        pltpu.VMEM((EG, 128), jnp.int32),
        pltpu.VMEM((EG, 128), jnp.int32),
        pltpu.VMEM((128, D), jnp.float32),
        pltpu.VMEM((128, L), jnp.float32),
        pltpu.VMEM((ROWS_PER_TILE, L), jnp.float32),
        pltpu.SemaphoreType.DMA,
        pltpu.VMEM_SHARED((N_PAD, D), jnp.float32),
        pltpu.VMEM_SHARED((N_PAD, L), jnp.float32),
    ],
)


def _dense_body(feat, a0, a1, w, b, p_out, q_out):
    w1a = w[0:D, :]
    w1b = w[D:2 * D, :]
    p_out[...] = jnp.dot(feat[...], w1a, preferred_element_type=jnp.float32) + b[...]
    q_out[...] = jnp.dot(a0[...] + a1[...], w1b, preferred_element_type=jnp.float32)


_DENSE_R = 1280

_dense_kernel = pl.pallas_call(
    _dense_body,
    grid=(N_PAD // _DENSE_R,),
    in_specs=[
        pl.BlockSpec((_DENSE_R, D), lambda i: (i, 0)),
        pl.BlockSpec((_DENSE_R, D), lambda i: (i, 0)),
        pl.BlockSpec((_DENSE_R, D), lambda i: (i + N_PAD // _DENSE_R, 0)),
        pl.BlockSpec((2 * D, D), lambda i: (0, 0)),
        pl.BlockSpec((1, D), lambda i: (0, 0)),
    ],
    out_specs=[
        pl.BlockSpec((_DENSE_R, D), lambda i: (i, 0)),
        pl.BlockSpec((_DENSE_R, D), lambda i: (i, 0)),
    ],
    out_shape=[
        jax.ShapeDtypeStruct((N_PAD, D), jnp.float32),
        jax.ShapeDtypeStruct((N_PAD, D), jnp.float32),
    ],
)


def _gather_body(p_hbm, q_hbm, deg_hbm, nidx_hbm, out_hbm,
                 nib, ni2b, pb, qb, d0b, d1b, sem):
    cid = lax.axis_index("c")
    sid = lax.axis_index("s")
    wid = sid * NC + cid

    pltpu.sync_copy(nidx_hbm.at[wid], nib)

    def _shift(i, _):
        g = i // 8
        c = (i % 8) * L
        ni2b[g, pl.ds(c, L)] = nib[g, pl.ds(c, L)] + N_PAD
        return 0
    lax.fori_loop(0, BG * 8, _shift, 0)

    def _group(g, _):
        cp = pltpu.async_copy(p_hbm.at[nib.at[g]], pb, sem)
        cq = pltpu.async_copy(q_hbm.at[nib.at[g]], qb, sem)
        c0 = pltpu.async_copy(deg_hbm.at[nib.at[g]], d0b, sem)
        c1 = pltpu.async_copy(deg_hbm.at[ni2b.at[g]], d1b, sem)
        cp.wait()
        cq.wait()
        c0.wait()
        c1.wait()

        def _rows(r, _):
            dv = jnp.maximum(d0b[r, :] + d1b[r, :], 1.0)
            for j in range(D // L):
                s = pl.ds(j * L, L)
                pb[r, s] = jnp.maximum(pb[r, s] + qb[r, s] / dv, 0.0)
            return 0
        lax.fori_loop(0, 128, _rows, 0)

        base = wid * (BG * 128) + g * 128
        pltpu.sync_copy(pb, out_hbm.at[pl.ds(base, 128)])
        return 0
    lax.fori_loop(0, BG, _group, 0)


_gather_kernel = pl.kernel(
    _gather_body,
    out_type=jax.ShapeDtypeStruct((B_PAD, D), jnp.float32),
    mesh=_MESH,
    scratch_types=[
        pltpu.VMEM((BG, 128), jnp.int32),
        pltpu.VMEM((BG, 128), jnp.int32),
        pltpu.VMEM((128, D), jnp.float32),
        pltpu.VMEM((128, D), jnp.float32),
        pltpu.VMEM((128, L), jnp.float32),
        pltpu.VMEM((128, L), jnp.float32),
        pltpu.SemaphoreType.DMA,
    ],
)


@jax.jit
def kernel(nodes, edge_index, feat_table, W1, b1):
    src = edge_index[0].astype(jnp.int32)
    dst = edge_index[1].astype(jnp.int32)
    src_p = jnp.concatenate(
        [src, jnp.zeros((E_PAD - E,), jnp.int32)]).reshape(NW, EG, 128)
    dst_p = jnp.concatenate(
        [dst, jnp.full((E_PAD - E,), N_PAD - 1, jnp.int32)]).reshape(NW, EG, 128)
    agg, deg = _edge_kernel(src_p, dst_p, feat_table)
    p, q = _dense_kernel(feat_table, agg, agg, W1, b1.reshape(1, D))
    nodes_p = jnp.concatenate(
        [nodes.astype(jnp.int32), jnp.zeros((B_PAD - B,), jnp.int32)]
    ).reshape(NW, BG, 128)
    outp = _gather_kernel(p, q, deg, nodes_p)
    return outp[:B]


# speedup vs baseline: 3.6526x; 3.6526x over previous
import jax
import jax.numpy as jnp
from jax import lax
from jax.experimental import pallas as pl
from jax.experimental.pallas import tpu as pltpu
from jax.experimental.pallas import tpu_sc as plsc

N = 10000
D = 128
E = 320000
B = 10000

NC, NS, L = 2, 16, 16
NW = NC * NS
N_PAD = 10240
ROWS_PER_TILE = N_PAD // NS
E_PAD = NW * 80 * 128
EG = 80
B_PAD = NW * 3 * 128
BG = 3

_MESH = plsc.VectorSubcoreMesh(
    core_axis_name="c", subcore_axis_name="s", num_cores=NC, num_subcores=NS
)
_SC_PARAMS = pltpu.CompilerParams(use_tc_tiling_on_sc=False)


def _edge_body(src_hbm, dst_hbm, feat_hbm, agg_hbm, deg_hbm,
               srcb, dstb, rows, ones, zblk, sem, agg_sh, deg_sh):
    cid = lax.axis_index("c")
    sid = lax.axis_index("s")
    wid = sid * NC + cid
    row0 = sid * ROWS_PER_TILE

    zf = jnp.zeros((L,), jnp.float32)
    of = jnp.ones((L,), jnp.float32)

    def _zrows(i, _):
        r = i // 8
        c = (i % 8) * L
        rows[r, pl.ds(c, L)] = zf
        return 0
    lax.fori_loop(0, 128 * 8, _zrows, 0)

    def _zblk(i, _):
        zblk[i, :] = zf
        ones[i, :] = of
        return 0
    lax.fori_loop(0, 128, _zblk, 0)

    for k in range(ROWS_PER_TILE // 128):
        pltpu.sync_copy(rows, agg_sh.at[pl.ds(row0 + k * 128, 128)])
        pltpu.sync_copy(zblk, deg_sh.at[pl.ds(row0 + k * 128, 128)])
    plsc.subcore_barrier()

    def _stage(t, _):
        pltpu.sync_copy(src_hbm.at[wid, pl.ds(t * 8, 8)], srcb)
        pltpu.sync_copy(dst_hbm.at[wid, pl.ds(t * 8, 8)], dstb)

        def _edges(j, _):
            pltpu.async_copy(feat_hbm.at[srcb.at[j]], rows, sem).wait()
            pltpu.sync_copy(rows, agg_sh.at[dstb.at[j]], add=True)
            pltpu.sync_copy(ones, deg_sh.at[dstb.at[j]], add=True)
            return 0
        lax.fori_loop(0, 8, _edges, 0)
        return 0
    lax.fori_loop(0, EG // 8, _stage, 0)

    plsc.subcore_barrier()

    out0 = cid * N_PAD + row0
    pltpu.sync_copy(agg_sh.at[pl.ds(row0, ROWS_PER_TILE)],
                    agg_hbm.at[pl.ds(out0, ROWS_PER_TILE)])
    pltpu.sync_copy(deg_sh.at[pl.ds(row0, ROWS_PER_TILE)],
                    deg_hbm.at[pl.ds(out0, ROWS_PER_TILE)])


_edge_kernel = pl.kernel(
    _edge_body,
    out_type=(
        jax.ShapeDtypeStruct((NC * N_PAD, D), jnp.float32),
        jax.ShapeDtypeStruct((NC * N_PAD, L), jnp.float32),
    ),
    mesh=_MESH,
    scratch_types=[
        pltpu.VMEM((8, 128), jnp.int32),
        pltpu.VMEM((8, 128), jnp.int32),
        pltpu.VMEM((128, D), jnp.float32),
        pltpu.VMEM((128, L), jnp.float32),
        pltpu.VMEM((128, L), jnp.float32),
        pltpu.SemaphoreType.DMA,
        pltpu.VMEM_SHARED((N_PAD, D), jnp.float32),
        pltpu.VMEM_SHARED((N_PAD, L), jnp.float32),
    ],
    compiler_params=_SC_PARAMS,
)


def _dense_body(feat, a0, a1, w, b, p_out, q_out):
    w1a = w[0:D, :]
    w1b = w[D:2 * D, :]
    p_out[...] = jnp.dot(feat[...], w1a, preferred_element_type=jnp.float32) + b[...]
    q_out[...] = jnp.dot(a0[...] + a1[...], w1b, preferred_element_type=jnp.float32)


_DENSE_R = 1280

_dense_kernel = pl.pallas_call(
    _dense_body,
    grid=(N_PAD // _DENSE_R,),
    in_specs=[
        pl.BlockSpec((_DENSE_R, D), lambda i: (i, 0)),
        pl.BlockSpec((_DENSE_R, D), lambda i: (i, 0)),
        pl.BlockSpec((_DENSE_R, D), lambda i: (i + N_PAD // _DENSE_R, 0)),
        pl.BlockSpec((2 * D, D), lambda i: (0, 0)),
        pl.BlockSpec((1, D), lambda i: (0, 0)),
    ],
    out_specs=[
        pl.BlockSpec((_DENSE_R, D), lambda i: (i, 0)),
        pl.BlockSpec((_DENSE_R, D), lambda i: (i, 0)),
    ],
    out_shape=[
        jax.ShapeDtypeStruct((N_PAD, D), jnp.float32),
        jax.ShapeDtypeStruct((N_PAD, D), jnp.float32),
    ],
)


def _gather_body(p_hbm, q_hbm, deg_hbm, nidx_hbm, out_hbm,
                 nib, ni2b, pb, qb, d0b, d1b, sem):
    cid = lax.axis_index("c")
    sid = lax.axis_index("s")
    wid = sid * NC + cid

    pltpu.sync_copy(nidx_hbm.at[wid], nib)

    def _shift(i, _):
        g = i // 8
        c = (i % 8) * L
        ni2b[g, pl.ds(c, L)] = nib[g, pl.ds(c, L)] + N_PAD
        return 0
    lax.fori_loop(0, BG * 8, _shift, 0)

    def _group(g, _):
        cp = pltpu.async_copy(p_hbm.at[nib.at[g]], pb, sem)
        cq = pltpu.async_copy(q_hbm.at[nib.at[g]], qb, sem)
        c0 = pltpu.async_copy(deg_hbm.at[nib.at[g]], d0b, sem)
        c1 = pltpu.async_copy(deg_hbm.at[ni2b.at[g]], d1b, sem)
        cp.wait()
        cq.wait()
        c0.wait()
        c1.wait()

        def _rows(r, _):
            dv = jnp.maximum(d0b[r, :] + d1b[r, :], 1.0)
            for j in range(D // L):
                s = pl.ds(j * L, L)
                pb[r, s] = jnp.maximum(pb[r, s] + qb[r, s] / dv, 0.0)
            return 0
        lax.fori_loop(0, 128, _rows, 0)

        base = wid * (BG * 128) + g * 128
        pltpu.sync_copy(pb, out_hbm.at[pl.ds(base, 128)])
        return 0
    lax.fori_loop(0, BG, _group, 0)


_gather_kernel = pl.kernel(
    _gather_body,
    out_type=jax.ShapeDtypeStruct((B_PAD, D), jnp.float32),
    mesh=_MESH,
    scratch_types=[
        pltpu.VMEM((BG, 128), jnp.int32),
        pltpu.VMEM((BG, 128), jnp.int32),
        pltpu.VMEM((128, D), jnp.float32),
        pltpu.VMEM((128, D), jnp.float32),
        pltpu.VMEM((128, L), jnp.float32),
        pltpu.VMEM((128, L), jnp.float32),
        pltpu.SemaphoreType.DMA,
    ],
    compiler_params=_SC_PARAMS,
)


@jax.jit
def kernel(nodes, edge_index, feat_table, W1, b1):
    src = edge_index[0].astype(jnp.int32)
    dst = edge_index[1].astype(jnp.int32)
    src_p = jnp.concatenate(
        [src, jnp.zeros((E_PAD - E,), jnp.int32)]).reshape(NW, EG, 128)
    dst_p = jnp.concatenate(
        [dst, jnp.full((E_PAD - E,), N_PAD - 1, jnp.int32)]).reshape(NW, EG, 128)
    agg, deg = _edge_kernel(src_p, dst_p, feat_table)
    p, q = _dense_kernel(feat_table, agg, agg, W1, b1.reshape(1, D))
    nodes_p = jnp.concatenate(
        [nodes.astype(jnp.int32), jnp.zeros((B_PAD - B,), jnp.int32)]
    ).reshape(NW, BG, 128)
    outp = _gather_kernel(p, q, deg, nodes_p)
    return outp[:B]


# speedup vs baseline: 4.1332x; 1.1316x over previous
import jax
import jax.numpy as jnp
from jax import lax
from jax.experimental import pallas as pl
from jax.experimental.pallas import tpu as pltpu
from jax.experimental.pallas import tpu_sc as plsc

N = 10000
D = 128
E = 320000
B = 10000

NC, NS, L = 2, 16, 16
NW = NC * NS
N_PAD = 10240
ROWS_PER_TILE = N_PAD // NS
E_PAD = NW * 80 * 128
EG = 80
B_PAD = NW * 3 * 128
BG = 3

_MESH = plsc.VectorSubcoreMesh(
    core_axis_name="c", subcore_axis_name="s", num_cores=NC, num_subcores=NS
)
_SC_PARAMS = pltpu.CompilerParams(use_tc_tiling_on_sc=False)


def _edge_body(src_hbm, dst_hbm, feat_hbm, agg_hbm, deg_hbm,
               srcb, dstb, rows0, rows1, ones, zblk, semg, sems, semd,
               agg_sh, deg_sh):
    cid = lax.axis_index("c")
    sid = lax.axis_index("s")
    wid = sid * NC + cid
    row0 = sid * ROWS_PER_TILE

    zf = jnp.zeros((L,), jnp.float32)
    of = jnp.ones((L,), jnp.float32)

    def _zrows(i, _):
        r = i // 8
        c = (i % 8) * L
        rows0[r, pl.ds(c, L)] = zf
        return 0
    lax.fori_loop(0, 128 * 8, _zrows, 0)

    def _zblk(i, _):
        ones[i, :] = of
        return 0
    lax.fori_loop(0, 128, _zblk, 0)

    def _zblk2(i, _):
        zblk[i, :] = zf
        return 0
    lax.fori_loop(0, 64, _zblk2, 0)

    for k in range(ROWS_PER_TILE // 128):
        pltpu.sync_copy(rows0, agg_sh.at[pl.ds(row0 + k * 128, 128)])
    for k in range(ROWS_PER_TILE // 64):
        pltpu.sync_copy(zblk, deg_sh.at[pl.ds(row0 + k * 64, 64)])
    plsc.subcore_barrier()

    pltpu.sync_copy(src_hbm.at[wid, pl.ds(0, 8)], srcb)
    pltpu.sync_copy(dst_hbm.at[wid, pl.ds(0, 8)], dstb)

    def _stage(t, _):
        pltpu.async_copy(feat_hbm.at[srcb.at[0]], rows0, semg)
        for j in range(8):
            rc = rows0 if j % 2 == 0 else rows1
            rn = rows1 if j % 2 == 0 else rows0
            if j < 7:
                if j >= 1:
                    pltpu.make_async_copy(
                        rn, agg_sh.at[dstb.at[j - 1]], sems).wait()
                pltpu.async_copy(feat_hbm.at[srcb.at[j + 1]], rn, semg)
            pltpu.make_async_copy(feat_hbm.at[srcb.at[j]], rc, semg).wait()
            pltpu.async_copy(rc, agg_sh.at[dstb.at[j]], sems, add=True)
            pltpu.async_copy(ones, deg_sh.at[dstb.at[j]], semd, add=True)
        pltpu.make_async_copy(rows0, agg_sh.at[dstb.at[6]], sems).wait()
        pltpu.make_async_copy(rows1, agg_sh.at[dstb.at[7]], sems).wait()
        for j in range(8):
            pltpu.make_async_copy(ones, deg_sh.at[dstb.at[j]], semd).wait()

        @pl.when(t < EG // 8 - 1)
        def _():
            pltpu.sync_copy(src_hbm.at[wid, pl.ds((t + 1) * 8, 8)], srcb)
            pltpu.sync_copy(dst_hbm.at[wid, pl.ds((t + 1) * 8, 8)], dstb)
        return 0
    lax.fori_loop(0, EG // 8, _stage, 0)

    plsc.subcore_barrier()

    out0 = cid * N_PAD + row0
    pltpu.sync_copy(agg_sh.at[pl.ds(row0, ROWS_PER_TILE)],
                    agg_hbm.at[pl.ds(out0, ROWS_PER_TILE)])
    pltpu.sync_copy(deg_sh.at[pl.ds(row0, ROWS_PER_TILE)],
                    deg_hbm.at[pl.ds(out0, ROWS_PER_TILE)])


_edge_kernel = pl.kernel(
    _edge_body,
    out_type=(
        jax.ShapeDtypeStruct((NC * N_PAD, D), jnp.float32),
        jax.ShapeDtypeStruct((NC * N_PAD, L), jnp.float32),
    ),
    mesh=_MESH,
    scratch_types=[
        pltpu.VMEM((8, 128), jnp.int32),
        pltpu.VMEM((8, 128), jnp.int32),
        pltpu.VMEM((128, D), jnp.float32),
        pltpu.VMEM((128, D), jnp.float32),
        pltpu.VMEM((128, L), jnp.float32),
        pltpu.VMEM((64, L), jnp.float32),
        pltpu.SemaphoreType.DMA,
        pltpu.SemaphoreType.DMA,
        pltpu.SemaphoreType.DMA,
        pltpu.VMEM_SHARED((N_PAD, D), jnp.float32),
        pltpu.VMEM_SHARED((N_PAD, L), jnp.float32),
    ],
    compiler_params=_SC_PARAMS,
)


def _dense_body(feat, a0, a1, w, b, p_out, q_out):
    w1a = w[0:D, :]
    w1b = w[D:2 * D, :]
    p_out[...] = jnp.dot(feat[...], w1a, preferred_element_type=jnp.float32) + b[...]
    q_out[...] = jnp.dot(a0[...] + a1[...], w1b, preferred_element_type=jnp.float32)


_DENSE_R = 1280

_dense_kernel = pl.pallas_call(
    _dense_body,
    grid=(N_PAD // _DENSE_R,),
    in_specs=[
        pl.BlockSpec((_DENSE_R, D), lambda i: (i, 0)),
        pl.BlockSpec((_DENSE_R, D), lambda i: (i, 0)),
        pl.BlockSpec((_DENSE_R, D), lambda i: (i + N_PAD // _DENSE_R, 0)),
        pl.BlockSpec((2 * D, D), lambda i: (0, 0)),
        pl.BlockSpec((1, D), lambda i: (0, 0)),
    ],
    out_specs=[
        pl.BlockSpec((_DENSE_R, D), lambda i: (i, 0)),
        pl.BlockSpec((_DENSE_R, D), lambda i: (i, 0)),
    ],
    out_shape=[
        jax.ShapeDtypeStruct((N_PAD, D), jnp.float32),
        jax.ShapeDtypeStruct((N_PAD, D), jnp.float32),
    ],
)


def _gather_body(p_hbm, q_hbm, deg_hbm, nidx_hbm, out_hbm,
                 nib, ni2b, pb, qb, d0b, d1b, sem):
    cid = lax.axis_index("c")
    sid = lax.axis_index("s")
    wid = sid * NC + cid

    pltpu.sync_copy(nidx_hbm.at[wid], nib)

    def _shift(i, _):
        g = i // 8
        c = (i % 8) * L
        ni2b[g, pl.ds(c, L)] = nib[g, pl.ds(c, L)] + N_PAD
        return 0
    lax.fori_loop(0, BG * 8, _shift, 0)

    def _group(g, _):
        cp = pltpu.async_copy(p_hbm.at[nib.at[g]], pb, sem)
        cq = pltpu.async_copy(q_hbm.at[nib.at[g]], qb, sem)
        c0 = pltpu.async_copy(deg_hbm.at[nib.at[g]], d0b, sem)
        c1 = pltpu.async_copy(deg_hbm.at[ni2b.at[g]], d1b, sem)
        cp.wait()
        cq.wait()
        c0.wait()
        c1.wait()

        def _rows(r, _):
            dv = jnp.maximum(d0b[r, :] + d1b[r, :], 1.0)
            for j in range(D // L):
                s = pl.ds(j * L, L)
                pb[r, s] = jnp.maximum(pb[r, s] + qb[r, s] / dv, 0.0)
            return 0
        lax.fori_loop(0, 128, _rows, 0)

        base = wid * (BG * 128) + g * 128
        pltpu.sync_copy(pb, out_hbm.at[pl.ds(base, 128)])
        return 0
    lax.fori_loop(0, BG, _group, 0)


_gather_kernel = pl.kernel(
    _gather_body,
    out_type=jax.ShapeDtypeStruct((B_PAD, D), jnp.float32),
    mesh=_MESH,
    scratch_types=[
        pltpu.VMEM((BG, 128), jnp.int32),
        pltpu.VMEM((BG, 128), jnp.int32),
        pltpu.VMEM((128, D), jnp.float32),
        pltpu.VMEM((128, D), jnp.float32),
        pltpu.VMEM((128, L), jnp.float32),
        pltpu.VMEM((128, L), jnp.float32),
        pltpu.SemaphoreType.DMA,
    ],
    compiler_params=_SC_PARAMS,
)


@jax.jit
def kernel(nodes, edge_index, feat_table, W1, b1):
    src = edge_index[0].astype(jnp.int32)
    dst = edge_index[1].astype(jnp.int32)
    src_p = jnp.concatenate(
        [src, jnp.zeros((E_PAD - E,), jnp.int32)]).reshape(NW, EG, 128)
    dst_p = jnp.concatenate(
        [dst, jnp.full((E_PAD - E,), N_PAD - 1, jnp.int32)]).reshape(NW, EG, 128)
    agg, deg = _edge_kernel(src_p, dst_p, feat_table)
    p, q = _dense_kernel(feat_table, agg, agg, W1, b1.reshape(1, D))
    nodes_p = jnp.concatenate(
        [nodes.astype(jnp.int32), jnp.zeros((B_PAD - B,), jnp.int32)]
    ).reshape(NW, BG, 128)
    outp = _gather_kernel(p, q, deg, nodes_p)
    return outp[:B]


# speedup vs baseline: 4.1380x; 1.0012x over previous
import jax
import jax.numpy as jnp
from jax import lax
from jax.experimental import pallas as pl
from jax.experimental.pallas import tpu as pltpu
from jax.experimental.pallas import tpu_sc as plsc

N = 10000
D = 128
E = 320000
B = 10000

NC, NS, L = 2, 16, 16
NW = NC * NS
N_PAD = 10240
ROWS_PER_TILE = N_PAD // NS
E_PAD = NW * 80 * 128
EG = 80
B_PAD = NW * 3 * 128
BG = 3

_MESH = plsc.VectorSubcoreMesh(
    core_axis_name="c", subcore_axis_name="s", num_cores=NC, num_subcores=NS
)
_SC_PARAMS = pltpu.CompilerParams(use_tc_tiling_on_sc=False)


def _edge_body(src_hbm, dst_hbm, feat_hbm, agg_hbm, deg_hbm,
               srcb, dstb, rows0, rows1, ones, zblk, semg, sems, semd,
               agg_sh, deg_sh):
    cid = lax.axis_index("c")
    sid = lax.axis_index("s")
    wid = sid * NC + cid
    row0 = sid * ROWS_PER_TILE

    zf = jnp.zeros((L,), jnp.float32)
    of = jnp.ones((L,), jnp.float32)

    def _zrows(i, _):
        r = i // 8
        c = (i % 8) * L
        rows0[r, pl.ds(c, L)] = zf
        return 0
    lax.fori_loop(0, 128 * 8, _zrows, 0)

    def _zblk(i, _):
        ones[i, :] = of
        return 0
    lax.fori_loop(0, 128, _zblk, 0)

    def _zblk2(i, _):
        zblk[i, :] = zf
        return 0
    lax.fori_loop(0, 64, _zblk2, 0)

    for k in range(ROWS_PER_TILE // 128):
        pltpu.sync_copy(rows0, agg_sh.at[pl.ds(row0 + k * 128, 128)])
    for k in range(ROWS_PER_TILE // 64):
        pltpu.sync_copy(zblk, deg_sh.at[pl.ds(row0 + k * 64, 64)])
    plsc.subcore_barrier()

    pltpu.sync_copy(src_hbm.at[wid, pl.ds(0, 8)], srcb)
    pltpu.sync_copy(dst_hbm.at[wid, pl.ds(0, 8)], dstb)

    def _stage(t, _):
        pltpu.async_copy(feat_hbm.at[srcb.at[0]], rows0, semg)
        for j in range(8):
            rc = rows0 if j % 2 == 0 else rows1
            rn = rows1 if j % 2 == 0 else rows0
            if j < 7:
                if j >= 1:
                    pltpu.make_async_copy(
                        rn, agg_sh.at[dstb.at[j - 1]], sems).wait()
                pltpu.async_copy(feat_hbm.at[srcb.at[j + 1]], rn, semg)
            pltpu.make_async_copy(feat_hbm.at[srcb.at[j]], rc, semg).wait()
            pltpu.async_copy(rc, agg_sh.at[dstb.at[j]], sems, add=True)
            pltpu.async_copy(ones, deg_sh.at[dstb.at[j]], semd, add=True)
        pltpu.make_async_copy(rows0, agg_sh.at[dstb.at[6]], sems).wait()
        pltpu.make_async_copy(rows1, agg_sh.at[dstb.at[7]], sems).wait()
        for j in range(8):
            pltpu.make_async_copy(ones, deg_sh.at[dstb.at[j]], semd).wait()

        @pl.when(t < EG // 8 - 1)
        def _():
            pltpu.sync_copy(src_hbm.at[wid, pl.ds((t + 1) * 8, 8)], srcb)
            pltpu.sync_copy(dst_hbm.at[wid, pl.ds((t + 1) * 8, 8)], dstb)
        return 0
    lax.fori_loop(0, EG // 8, _stage, 0)

    plsc.subcore_barrier()

    out0 = cid * N_PAD + row0
    pltpu.sync_copy(agg_sh.at[pl.ds(row0, ROWS_PER_TILE)],
                    agg_hbm.at[pl.ds(out0, ROWS_PER_TILE)])
    pltpu.sync_copy(deg_sh.at[pl.ds(row0, ROWS_PER_TILE)],
                    deg_hbm.at[pl.ds(out0, ROWS_PER_TILE)])


_edge_kernel = pl.kernel(
    _edge_body,
    out_type=(
        jax.ShapeDtypeStruct((NC * N_PAD, D), jnp.float32),
        jax.ShapeDtypeStruct((NC * N_PAD, L), jnp.float32),
    ),
    mesh=_MESH,
    scratch_types=[
        pltpu.VMEM((8, 128), jnp.int32),
        pltpu.VMEM((8, 128), jnp.int32),
        pltpu.VMEM((128, D), jnp.float32),
        pltpu.VMEM((128, D), jnp.float32),
        pltpu.VMEM((128, L), jnp.float32),
        pltpu.VMEM((64, L), jnp.float32),
        pltpu.SemaphoreType.DMA,
        pltpu.SemaphoreType.DMA,
        pltpu.SemaphoreType.DMA,
        pltpu.VMEM_SHARED((N_PAD, D), jnp.float32),
        pltpu.VMEM_SHARED((N_PAD, L), jnp.float32),
    ],
    compiler_params=_SC_PARAMS,
)


def _dense_body(feat, a0, a1, w, b, p_out, q_out):
    w1a = w[0:D, :]
    w1b = w[D:2 * D, :]
    p_out[...] = jnp.dot(feat[...], w1a, preferred_element_type=jnp.float32) + b[...]
    q_out[...] = jnp.dot(a0[...] + a1[...], w1b, preferred_element_type=jnp.float32)


_DENSE_R = 1280

_dense_kernel = pl.pallas_call(
    _dense_body,
    grid=(N_PAD // _DENSE_R,),
    in_specs=[
        pl.BlockSpec((_DENSE_R, D), lambda i: (i, 0)),
        pl.BlockSpec((_DENSE_R, D), lambda i: (i, 0)),
        pl.BlockSpec((_DENSE_R, D), lambda i: (i + N_PAD // _DENSE_R, 0)),
        pl.BlockSpec((2 * D, D), lambda i: (0, 0)),
        pl.BlockSpec((1, D), lambda i: (0, 0)),
    ],
    out_specs=[
        pl.BlockSpec((_DENSE_R, D), lambda i: (i, 0)),
        pl.BlockSpec((_DENSE_R, D), lambda i: (i, 0)),
    ],
    out_shape=[
        jax.ShapeDtypeStruct((N_PAD, D), jnp.float32),
        jax.ShapeDtypeStruct((N_PAD, D), jnp.float32),
    ],
)


def _gather_body(p_hbm, q_hbm, deg_hbm, nidx_hbm, out_hbm,
                 nib, ni2b, pb, qb, d0b, d1b, sem):
    cid = lax.axis_index("c")
    sid = lax.axis_index("s")
    wid = sid * NC + cid

    pltpu.sync_copy(nidx_hbm.at[wid], nib)

    def _shift(i, _):
        g = i // 8
        c = (i % 8) * L
        ni2b[g, pl.ds(c, L)] = nib[g, pl.ds(c, L)] + N_PAD
        return 0
    lax.fori_loop(0, BG * 8, _shift, 0)

    def _group(g, _):
        cp = pltpu.async_copy(p_hbm.at[nib.at[g]], pb, sem)
        cq = pltpu.async_copy(q_hbm.at[nib.at[g]], qb, sem)
        c0 = pltpu.async_copy(deg_hbm.at[nib.at[g]], d0b, sem)
        c1 = pltpu.async_copy(deg_hbm.at[ni2b.at[g]], d1b, sem)
        cp.wait()
        cq.wait()
        c0.wait()
        c1.wait()

        def _rows(r, _):
            dv = jnp.maximum(d0b[r, :] + d1b[r, :], 1.0)
            for j in range(D // L):
                s = pl.ds(j * L, L)
                pb[r, s] = jnp.maximum(pb[r, s] + qb[r, s] / dv, 0.0)
            return 0
        lax.fori_loop(0, 128, _rows, 0)

        base = wid * (BG * 128) + g * 128
        pltpu.sync_copy(pb, out_hbm.at[pl.ds(base, 128)])
        return 0
    lax.fori_loop(0, BG, _group, 0)


_gather_kernel = pl.kernel(
    _gather_body,
    out_type=jax.ShapeDtypeStruct((B_PAD, D), jnp.float32),
    mesh=_MESH,
    scratch_types=[
        pltpu.VMEM((BG, 128), jnp.int32),
        pltpu.VMEM((BG, 128), jnp.int32),
        pltpu.VMEM((128, D), jnp.float32),
        pltpu.VMEM((128, D), jnp.float32),
        pltpu.VMEM((128, L), jnp.float32),
        pltpu.VMEM((128, L), jnp.float32),
        pltpu.SemaphoreType.DMA,
    ],
    compiler_params=_SC_PARAMS,
)


@jax.jit
def kernel(nodes, edge_index, feat_table, W1, b1):
    src = edge_index[0].astype(jnp.int32)
    dst = edge_index[1].astype(jnp.int32)
    src_p = jnp.concatenate(
        [src, jnp.zeros((E_PAD - E,), jnp.int32)]).reshape(NW, EG, 128)
    pad_dst = N + jax.lax.rem(jnp.arange(E_PAD - E, dtype=jnp.int32),
                              jnp.int32(N_PAD - N))
    dst_p = jnp.concatenate([dst, pad_dst]).reshape(NW, EG, 128)
    agg, deg = _edge_kernel(src_p, dst_p, feat_table)
    p, q = _dense_kernel(feat_table, agg, agg, W1, b1.reshape(1, D))
    nodes_p = jnp.concatenate(
        [nodes.astype(jnp.int32), jnp.zeros((B_PAD - B,), jnp.int32)]
    ).reshape(NW, BG, 128)
    outp = _gather_kernel(p, q, deg, nodes_p)
    return outp[:B]
